# baseline (speedup 1.0000x reference)
import jax
import jax.numpy as jnp
from jax.experimental import pallas as pl
from jax.experimental.pallas import tpu as pltpu

_P = 2


def _med3(a, b, c):
    return jnp.maximum(jnp.minimum(a, b), jnp.minimum(jnp.maximum(a, b), c))


def _median3x3_kernel(x_ref, o_ref, s_lo, s_mid, s_hi, s_out):
    P, H, W = x_ref.shape
    nh = W // 128
    dt = o_ref.dtype
    zcol = jnp.zeros((H, 1), dt)
    zrow = jnp.zeros((1, 128), dt)

    for p in range(P):
        x = x_ref[p]
        xl = jnp.concatenate([zcol, x[:, :-1]], axis=1)
        xr = jnp.concatenate([x[:, 1:], zcol], axis=1)

        mnh = jnp.minimum(x, xr)
        mxh = jnp.maximum(x, xr)
        lo = jnp.minimum(xl, mnh)
        hi = jnp.maximum(xl, mxh)
        mid = jnp.maximum(jnp.minimum(xl, mxh), mnh)

        for h in range(nh):
            cs = slice(128 * h, 128 * (h + 1))
            s_lo[h] = lo[:, cs]
            s_mid[h] = mid[:, cs]
            s_hi[h] = hi[:, cs]

        for h in range(nh):
            lop = [s_lo[h, r:H:8, :] for r in range(8)]
            midp = [s_mid[h, r:H:8, :] for r in range(8)]
            hip = [s_hi[h, r:H:8, :] for r in range(8)]
            lo_u = jnp.concatenate([zrow, lop[7][:-1]], axis=0)
            mid_u = jnp.concatenate([zrow, midp[7][:-1]], axis=0)
            hi_u = jnp.concatenate([zrow, hip[7][:-1]], axis=0)
            lo_d = jnp.concatenate([lop[0][1:], zrow], axis=0)
            mid_d = jnp.concatenate([midp[0][1:], zrow], axis=0)
            hi_d = jnp.concatenate([hip[0][1:], zrow], axis=0)
            for r in range(8):
                ul, um, uh = (lo_u, mid_u, hi_u) if r == 0 else \
                    (lop[r - 1], midp[r - 1], hip[r - 1])
                dl, dm, dh = (lo_d, mid_d, hi_d) if r == 7 else \
                    (lop[r + 1], midp[r + 1], hip[r + 1])
                mx = jnp.maximum(jnp.maximum(ul, lop[r]), dl)
                mn = jnp.minimum(jnp.minimum(uh, hip[r]), dh)
                md = _med3(um, midp[r], dm)
                s_out[h, r:H:8, :] = _med3(mx, md, mn)

        for h in range(nh):
            o_ref[p, :, 128 * h:128 * (h + 1)] = s_out[h]


@jax.jit
def kernel(x):
    B, C, H, W = x.shape
    N = B * C
    xf = x.reshape(N, H, W)
    nh = W // 128
    scratch = [pltpu.VMEM((nh, H, 128), x.dtype) for _ in range(4)]
    out = pl.pallas_call(
        _median3x3_kernel,
        grid=(N // _P,),
        in_specs=[pl.BlockSpec((_P, H, W), lambda i: (i, 0, 0))],
        out_specs=pl.BlockSpec((_P, H, W), lambda i: (i, 0, 0)),
        out_shape=jax.ShapeDtypeStruct((N, H, W), x.dtype),
        scratch_shapes=scratch,
        compiler_params=pltpu.CompilerParams(
            dimension_semantics=("parallel",),
        ),
    )(xf)
    return out.reshape(B, C, H, W)


# speedup vs baseline: 1.0500x; 1.0500x over previous
import jax
import jax.numpy as jnp
from jax.experimental import pallas as pl
from jax.experimental.pallas import tpu as pltpu

_P = 2
_CH = 4


def _med3(a, b, c):
    return jnp.maximum(jnp.minimum(a, b), jnp.minimum(jnp.maximum(a, b), c))


def _median3x3_kernel(x_ref, o_ref):
    P, H, W = x_ref.shape
    Wc = W // _CH
    dt = o_ref.dtype

    for p in range(P):
        x = x_ref[p]
        zcol = jnp.zeros((H, 1), dt)
        xl = jnp.concatenate([zcol, x[:, :-1]], axis=1)
        xr = jnp.concatenate([x[:, 1:], zcol], axis=1)

        mnh = jnp.minimum(x, xr)
        mxh = jnp.maximum(x, xr)
        lo = jnp.minimum(xl, mnh)
        hi = jnp.maximum(xl, mxh)
        mid = jnp.maximum(jnp.minimum(xl, mxh), mnh)

        zrow = jnp.zeros((1, Wc), dt)

        def shu(a):
            return jnp.concatenate([zrow, a[:-1, :]], axis=0)

        def shd(a):
            return jnp.concatenate([a[1:, :], zrow], axis=0)

        for c in range(_CH):
            cs = slice(Wc * c, Wc * (c + 1))
            loc, midc, hic = lo[:, cs], mid[:, cs], hi[:, cs]
            mx = jnp.maximum(jnp.maximum(shu(loc), loc), shd(loc))
            mn = jnp.minimum(jnp.minimum(shu(hic), hic), shd(hic))
            md = _med3(shu(midc), midc, shd(midc))
            o_ref[p, :, cs] = _med3(mx, md, mn)


@jax.jit
def kernel(x):
    B, C, H, W = x.shape
    N = B * C
    xf = x.reshape(N, H, W)
    out = pl.pallas_call(
        _median3x3_kernel,
        grid=(N // _P,),
        in_specs=[pl.BlockSpec((_P, H, W), lambda i: (i, 0, 0))],
        out_specs=pl.BlockSpec((_P, H, W), lambda i: (i, 0, 0)),
        out_shape=jax.ShapeDtypeStruct((N, H, W), x.dtype),
        compiler_params=pltpu.CompilerParams(
            dimension_semantics=("parallel",),
        ),
    )(xf)
    return out.reshape(B, C, H, W)


# speedup vs baseline: 1.0623x; 1.0118x over previous
import jax
import jax.numpy as jnp
from jax.experimental import pallas as pl
from jax.experimental.pallas import tpu as pltpu

_P = 4
_CH = 4


def _med3(a, b, c):
    return jnp.maximum(jnp.minimum(a, b), jnp.minimum(jnp.maximum(a, b), c))


def _median3x3_kernel(x_ref, o_ref):
    P, H, W = x_ref.shape
    Wc = W // _CH
    dt = o_ref.dtype

    for p in range(P):
        x = x_ref[p]
        zcol = jnp.zeros((H, 1), dt)
        xl = jnp.concatenate([zcol, x[:, :-1]], axis=1)
        xr = jnp.concatenate([x[:, 1:], zcol], axis=1)

        mnh = jnp.minimum(x, xr)
        mxh = jnp.maximum(x, xr)
        lo = jnp.minimum(xl, mnh)
        hi = jnp.maximum(xl, mxh)
        mid = jnp.maximum(jnp.minimum(xl, mxh), mnh)

        zrow = jnp.zeros((1, Wc), dt)

        def shu(a):
            return jnp.concatenate([zrow, a[:-1, :]], axis=0)

        def shd(a):
            return jnp.concatenate([a[1:, :], zrow], axis=0)

        for c in range(_CH):
            cs = slice(Wc * c, Wc * (c + 1))
            loc, midc, hic = lo[:, cs], mid[:, cs], hi[:, cs]
            mx = jnp.maximum(jnp.maximum(shu(loc), loc), shd(loc))
            mn = jnp.minimum(jnp.minimum(shu(hic), hic), shd(hic))
            md = _med3(shu(midc), midc, shd(midc))
            o_ref[p, :, cs] = _med3(mx, md, mn)


@jax.jit
def kernel(x):
    B, C, H, W = x.shape
    N = B * C
    xf = x.reshape(N, H, W)
    out = pl.pallas_call(
        _median3x3_kernel,
        grid=(N // _P,),
        in_specs=[pl.BlockSpec((_P, H, W), lambda i: (i, 0, 0))],
        out_specs=pl.BlockSpec((_P, H, W), lambda i: (i, 0, 0)),
        out_shape=jax.ShapeDtypeStruct((N, H, W), x.dtype),
        compiler_params=pltpu.CompilerParams(
            dimension_semantics=("parallel",),
        ),
    )(xf)
    return out.reshape(B, C, H, W)
